# Initial kernel scaffold; baseline (speedup 1.0000x reference)
import jax
import jax.numpy as jnp
from jax.experimental import pallas as pl


def kernel(nodes, edges, neighbor_idxs, params):
    raise NotImplementedError("write your pallas kernel here")


# speedup vs baseline: 2.9696x; 2.9696x over previous
import functools

import jax
import jax.numpy as jnp
from jax import lax
from jax.experimental import pallas as pl
from jax.experimental.pallas import tpu as pltpu
from jax.experimental.pallas import tpu_sc as plsc

_NC = 2
_NS = 16
_NW = _NC * _NS

_GB = 80

_F32 = jnp.float32


def _gather_kernel(E, D):
    ew = E // _NW
    gc = ew // _GB
    mesh = plsc.VectorSubcoreMesh(core_axis_name="c", subcore_axis_name="s")

    @functools.partial(
        pl.kernel,
        out_type=[jax.ShapeDtypeStruct((E, D), _F32),
                  jax.ShapeDtypeStruct((E, D), _F32)],
        mesh=mesh,
        scratch_types=[
            pltpu.VMEM((gc, _GB), jnp.int32),
            pltpu.VMEM((gc, _GB), jnp.int32),
            pltpu.VMEM((_GB, D), _F32),
            pltpu.VMEM((_GB, D), _F32),
            pltpu.SemaphoreType.DMA,
            pltpu.SemaphoreType.DMA,
        ],
    )
    def gather(nodes_h, idx0_h, idx1_h, recv_h, send_h,
               idx0_v, idx1_v, buf_a, buf_b, sem_a, sem_b):
        c = lax.axis_index("c")
        s = lax.axis_index("s")
        w = s * _NC + c
        pltpu.sync_copy(idx0_h.at[w], idx0_v)
        pltpu.sync_copy(idx1_h.at[w], idx1_v)

        def chunk(i, carry):
            a = pltpu.async_copy(nodes_h.at[idx0_v.at[i]], buf_a, sem_a)
            b = pltpu.async_copy(nodes_h.at[idx1_v.at[i]], buf_b, sem_b)
            a.wait()
            b.wait()
            base = w * ew + i * _GB
            pltpu.sync_copy(buf_a, recv_h.at[pl.ds(base, _GB), :])
            pltpu.sync_copy(buf_b, send_h.at[pl.ds(base, _GB), :])
            return carry

        lax.fori_loop(0, gc, chunk, 0)

    return gather


def _scatter_kernel(E, NP, D):
    ew = E // _NW
    gc = ew // _GB
    nt = NP // _NS
    mesh = plsc.VectorSubcoreMesh(core_axis_name="c", subcore_axis_name="s")

    @functools.partial(
        pl.kernel,
        out_type=jax.ShapeDtypeStruct((_NC, NP, D), _F32),
        mesh=mesh,
        scratch_types=[
            pltpu.VMEM((gc, _GB), jnp.int32),
            pltpu.VMEM((_GB, D), _F32),
            pltpu.VMEM_SHARED((NP, D), _F32),
        ],
    )
    def scatter(edges_h, idx0_h, zeros_h, out_h, idx_v, ebuf, agg_sh):
        c = lax.axis_index("c")
        s = lax.axis_index("s")
        w = s * _NC + c
        pltpu.sync_copy(zeros_h.at[pl.ds(s * nt, nt), :],
                        agg_sh.at[pl.ds(s * nt, nt), :])
        pltpu.sync_copy(idx0_h.at[w], idx_v)
        plsc.subcore_barrier()

        def chunk(i, carry):
            base = w * ew + i * _GB
            pltpu.sync_copy(edges_h.at[pl.ds(base, _GB), :], ebuf)
            pltpu.sync_copy(ebuf, agg_sh.at[idx_v.at[i]], add=True)
            return carry

        lax.fori_loop(0, gc, chunk, 0)
        plsc.subcore_barrier()
        pltpu.sync_copy(agg_sh.at[pl.ds(s * nt, nt), :],
                        out_h.at[c, pl.ds(s * nt, nt), :])

    return scatter


def _edge_mlp_body(ed, rv, sd, w0e, w0r, w0s, b0, w1, b1, w2, b2, g, bn, out):
    x = jnp.dot(ed[...], w0e[...], preferred_element_type=_F32)
    x += jnp.dot(rv[...], w0r[...], preferred_element_type=_F32)
    x += jnp.dot(sd[...], w0s[...], preferred_element_type=_F32)
    x = jax.nn.relu(x + b0[...])
    x = jax.nn.relu(jnp.dot(x, w1[...], preferred_element_type=_F32) + b1[...])
    y = jnp.dot(x, w2[...], preferred_element_type=_F32) + b2[...]
    mu = jnp.mean(y, axis=-1, keepdims=True)
    var = jnp.mean((y - mu) ** 2, axis=-1, keepdims=True)
    out[...] = (y - mu) * lax.rsqrt(var + 1e-5) * g[...] + bn[...]


def _node_mlp_body(p0, p1, nd, w0a, w0n, b0, w1, b1, w2, b2, g, bn, out):
    agg = p0[...] + p1[...]
    x = jnp.dot(agg, w0a[...], preferred_element_type=_F32)
    x += jnp.dot(nd[...], w0n[...], preferred_element_type=_F32)
    x = jax.nn.relu(x + b0[...])
    x = jax.nn.relu(jnp.dot(x, w1[...], preferred_element_type=_F32) + b1[...])
    y = jnp.dot(x, w2[...], preferred_element_type=_F32) + b2[...]
    mu = jnp.mean(y, axis=-1, keepdims=True)
    var = jnp.mean((y - mu) ** 2, axis=-1, keepdims=True)
    out[...] = (y - mu) * lax.rsqrt(var + 1e-5) * g[...] + bn[...]


def _row_spec(tile, d):
    return pl.BlockSpec((tile, d), lambda i: (i, 0))


def _full_spec(shape):
    return pl.BlockSpec(shape, lambda i: (0,) * len(shape))


def _edge_mlp_call(E, D, H, tile):
    grid = (E // tile,)
    w = _full_spec((D, H))
    wh = _full_spec((H, H))
    w2 = _full_spec((H, D))
    b = _full_spec((1, H))
    bd = _full_spec((1, D))
    return pl.pallas_call(
        _edge_mlp_body,
        grid=grid,
        in_specs=[_row_spec(tile, D)] * 3 + [w, w, w, b, wh, b, w2, bd, bd, bd],
        out_specs=_row_spec(tile, D),
        out_shape=jax.ShapeDtypeStruct((E, D), _F32),
    )


def _node_mlp_call(N, D, H, tile):
    grid = (N // tile,)
    w = _full_spec((D, H))
    wh = _full_spec((H, H))
    w2 = _full_spec((H, D))
    b = _full_spec((1, H))
    bd = _full_spec((1, D))
    return pl.pallas_call(
        _node_mlp_body,
        grid=grid,
        in_specs=[_row_spec(tile, D)] * 3 + [w, w, b, wh, b, w2, bd, bd, bd],
        out_specs=_row_spec(tile, D),
        out_shape=jax.ShapeDtypeStruct((N, D), _F32),
    )


def kernel(nodes, edges, neighbor_idxs, params):
    N, D = nodes.shape
    E = edges.shape[0]
    L = params["edge_W0"].shape[0]
    H = params["edge_W1"].shape[1]

    ew = E // _NW
    gc = ew // _GB
    idx0 = neighbor_idxs[:, 0].astype(jnp.int32).reshape(_NW, gc, _GB)
    idx1 = neighbor_idxs[:, 1].astype(jnp.int32).reshape(_NW, gc, _GB)
    NP = ((N + 8 * _NS - 1) // (8 * _NS)) * (8 * _NS)
    zeros = jnp.zeros((NP, D), _F32)

    gather = _gather_kernel(E, D)
    scatter = _scatter_kernel(E, NP, D)
    edge_mlp = _edge_mlp_call(E, D, H, tile=2000)
    node_mlp = _node_mlp_call(N, D, H, tile=2000)

    def b2d(b):
        return b.reshape(1, -1)

    for i in range(L):
        recv, send = gather(nodes, idx0, idx1)
        ew0 = params["edge_W0"][i]
        edges = edge_mlp(
            edges, recv, send,
            ew0[:D], ew0[D:2 * D], ew0[2 * D:],
            b2d(params["edge_b0"][i]), params["edge_W1"][i],
            b2d(params["edge_b1"][i]), params["edge_W2"][i],
            b2d(params["edge_b2"][i]), b2d(params["edge_ln_g"][i]),
            b2d(params["edge_ln_b"][i]))
        parts = scatter(edges, idx0, zeros)
        nw0 = params["node_W0"][i]
        nodes = node_mlp(
            parts[0, :N], parts[1, :N], nodes,
            nw0[:D], nw0[D:],
            b2d(params["node_b0"][i]), params["node_W1"][i],
            b2d(params["node_b1"][i]), params["node_W2"][i],
            b2d(params["node_b2"][i]), b2d(params["node_ln_g"][i]),
            b2d(params["node_ln_b"][i]))

    return nodes, edges, neighbor_idxs


# speedup vs baseline: 3.1652x; 1.0659x over previous
import functools

import jax
import jax.numpy as jnp
from jax import lax
from jax.experimental import pallas as pl
from jax.experimental.pallas import tpu as pltpu
from jax.experimental.pallas import tpu_sc as plsc

_NC = 2
_NS = 16
_NW = _NC * _NS

_GB = 80

_F32 = jnp.float32


_NB = 5


def _gather_kernel(E, D):
    ew = E // _NW
    gc = ew // _GB
    assert gc % _NB == 0 and gc >= 2 * _NB
    mesh = plsc.VectorSubcoreMesh(core_axis_name="c", subcore_axis_name="s")

    nt2 = 2 * gc

    @functools.partial(
        pl.kernel,
        out_type=[jax.ShapeDtypeStruct((E, D), _F32),
                  jax.ShapeDtypeStruct((E, D), _F32)],
        mesh=mesh,
        scratch_types=[
            pltpu.VMEM((gc, _GB), jnp.int32),
            pltpu.VMEM((gc, _GB), jnp.int32),
        ] + [pltpu.VMEM((_GB, D), _F32)] * _NB
          + [pltpu.SemaphoreType.DMA] * (2 * _NB),
    )
    def gather(nodes_h, idx0_h, idx1_h, recv_h, send_h, idx0_v, idx1_v, *rest):
        bufs = rest[0:_NB]
        gsem = rest[_NB:2 * _NB]
        wsem = rest[2 * _NB:3 * _NB]
        c = lax.axis_index("c")
        s = lax.axis_index("s")
        w = s * _NC + c
        base_e = w * ew
        pltpu.sync_copy(idx0_h.at[w], idx0_v)
        pltpu.sync_copy(idx1_h.at[w], idx1_v)

        def fire(ch, parity, slot):
            idx = idx0_v if parity == 0 else idx1_v
            pltpu.async_copy(nodes_h.at[idx.at[ch]], bufs[slot], gsem[slot])

        fire(0, 0, 0)
        fire(0, 1, 1)

        def body(i, carry):
            for p in range(2 * _NB):
                ch = i * _NB + p // 2
                sl = p % _NB
                sp = (p + 2) % _NB
                t = i * 2 * _NB + p

                def drain():
                    pltpu.make_async_copy(
                        bufs[sp], recv_h.at[pl.ds(base_e, _GB), :],
                        wsem[sp]).wait()

                if p >= 3:
                    drain()
                else:
                    pl.when(t >= 3)(drain)

                @pl.when(t + 2 < nt2)
                def _():
                    fire(ch + 1, p % 2, sp)

                pltpu.make_async_copy(
                    recv_h.at[pl.ds(base_e, _GB), :], bufs[sl],
                    gsem[sl]).wait()
                off = base_e + ch * _GB
                dst = recv_h if p % 2 == 0 else send_h
                pltpu.async_copy(bufs[sl], dst.at[pl.ds(off, _GB), :],
                                 wsem[sl])
            return carry

        lax.fori_loop(0, gc // _NB, body, 0)
        for t in (nt2 - 3, nt2 - 2, nt2 - 1):
            sp = t % _NB
            pltpu.make_async_copy(
                bufs[sp], recv_h.at[pl.ds(base_e, _GB), :], wsem[sp]).wait()

    return gather


_GBS = 40


def _scatter_kernel(E, NP, D):
    ew = E // _NW
    gc = ew // _GBS
    nt = NP // _NS
    mesh = plsc.VectorSubcoreMesh(core_axis_name="c", subcore_axis_name="s")

    assert gc % 10 == 0

    @functools.partial(
        pl.kernel,
        out_type=jax.ShapeDtypeStruct((_NC, NP, D), _F32),
        mesh=mesh,
        scratch_types=[
            pltpu.VMEM((gc, _GBS), jnp.int32),
            pltpu.VMEM_SHARED((NP, D), _F32),
        ] + [pltpu.VMEM((_GBS, D), _F32)] * 2
          + [pltpu.SemaphoreType.DMA] * 2,
    )
    def scatter(edges_h, idx0_h, zeros_h, out_h, idx_v, agg_sh, *rest):
        bufs = rest[0:2]
        rsem = rest[2:4]
        c = lax.axis_index("c")
        s = lax.axis_index("s")
        w = s * _NC + c
        base_e = w * ew
        pltpu.sync_copy(zeros_h.at[pl.ds(s * nt, nt), :],
                        agg_sh.at[pl.ds(s * nt, nt), :])
        pltpu.sync_copy(idx0_h.at[w], idx_v)
        plsc.subcore_barrier()

        pltpu.async_copy(edges_h.at[pl.ds(base_e, _GBS), :], bufs[0], rsem[0])

        def body(i, carry):
            for p in range(10):
                ch = i * 10 + p
                sl = p % 2
                sp = (p + 1) % 2
                pltpu.make_async_copy(
                    edges_h.at[pl.ds(base_e, _GBS), :], bufs[sl],
                    rsem[sl]).wait()

                @pl.when(ch + 1 < gc)
                def _():
                    pltpu.async_copy(
                        edges_h.at[pl.ds(base_e + (ch + 1) * _GBS, _GBS), :],
                        bufs[sp], rsem[sp])

                pltpu.sync_copy(bufs[sl], agg_sh.at[idx_v.at[ch]], add=True)
            return carry

        lax.fori_loop(0, gc // 10, body, 0)
        plsc.subcore_barrier()
        pltpu.sync_copy(agg_sh.at[pl.ds(s * nt, nt), :],
                        out_h.at[c, pl.ds(s * nt, nt), :])

    return scatter


def _edge_mlp_body(ed, rv, sd, w0e, w0r, w0s, b0, w1, b1, w2, b2, g, bn, out):
    x = jnp.dot(ed[...], w0e[...], preferred_element_type=_F32)
    x += jnp.dot(rv[...], w0r[...], preferred_element_type=_F32)
    x += jnp.dot(sd[...], w0s[...], preferred_element_type=_F32)
    x = jax.nn.relu(x + b0[...])
    x = jax.nn.relu(jnp.dot(x, w1[...], preferred_element_type=_F32) + b1[...])
    y = jnp.dot(x, w2[...], preferred_element_type=_F32) + b2[...]
    mu = jnp.mean(y, axis=-1, keepdims=True)
    var = jnp.mean((y - mu) ** 2, axis=-1, keepdims=True)
    out[...] = (y - mu) * lax.rsqrt(var + 1e-5) * g[...] + bn[...]


def _node_mlp_body(p0, p1, nd, w0a, w0n, b0, w1, b1, w2, b2, g, bn, out):
    agg = p0[...] + p1[...]
    x = jnp.dot(agg, w0a[...], preferred_element_type=_F32)
    x += jnp.dot(nd[...], w0n[...], preferred_element_type=_F32)
    x = jax.nn.relu(x + b0[...])
    x = jax.nn.relu(jnp.dot(x, w1[...], preferred_element_type=_F32) + b1[...])
    y = jnp.dot(x, w2[...], preferred_element_type=_F32) + b2[...]
    mu = jnp.mean(y, axis=-1, keepdims=True)
    var = jnp.mean((y - mu) ** 2, axis=-1, keepdims=True)
    out[...] = (y - mu) * lax.rsqrt(var + 1e-5) * g[...] + bn[...]


def _row_spec(tile, d):
    return pl.BlockSpec((tile, d), lambda i: (i, 0))


def _full_spec(shape):
    return pl.BlockSpec(shape, lambda i: (0,) * len(shape))


def _edge_mlp_call(E, D, H, tile):
    grid = (E // tile,)
    w = _full_spec((D, H))
    wh = _full_spec((H, H))
    w2 = _full_spec((H, D))
    b = _full_spec((1, H))
    bd = _full_spec((1, D))
    return pl.pallas_call(
        _edge_mlp_body,
        grid=grid,
        in_specs=[_row_spec(tile, D)] * 3 + [w, w, w, b, wh, b, w2, bd, bd, bd],
        out_specs=_row_spec(tile, D),
        out_shape=jax.ShapeDtypeStruct((E, D), _F32),
    )


def _node_mlp_call(N, D, H, tile):
    grid = (N // tile,)
    w = _full_spec((D, H))
    wh = _full_spec((H, H))
    w2 = _full_spec((H, D))
    b = _full_spec((1, H))
    bd = _full_spec((1, D))
    return pl.pallas_call(
        _node_mlp_body,
        grid=grid,
        in_specs=[_row_spec(tile, D)] * 3 + [w, w, b, wh, b, w2, bd, bd, bd],
        out_specs=_row_spec(tile, D),
        out_shape=jax.ShapeDtypeStruct((N, D), _F32),
    )


def kernel(nodes, edges, neighbor_idxs, params):
    N, D = nodes.shape
    E = edges.shape[0]
    L = params["edge_W0"].shape[0]
    H = params["edge_W1"].shape[1]

    ew = E // _NW
    gc = ew // _GB
    idx0 = neighbor_idxs[:, 0].astype(jnp.int32).reshape(_NW, gc, _GB)
    idx1 = neighbor_idxs[:, 1].astype(jnp.int32).reshape(_NW, gc, _GB)
    idx0_s = idx0.reshape(_NW, ew // _GBS, _GBS)
    NP = ((N + 8 * _NS - 1) // (8 * _NS)) * (8 * _NS)
    zeros = jnp.zeros((NP, D), _F32)

    gather = _gather_kernel(E, D)
    scatter = _scatter_kernel(E, NP, D)
    edge_mlp = _edge_mlp_call(E, D, H, tile=2000)
    node_mlp = _node_mlp_call(N, D, H, tile=2000)

    def b2d(b):
        return b.reshape(1, -1)

    for i in range(L):
        recv, send = gather(nodes, idx0, idx1)
        ew0 = params["edge_W0"][i]
        edges = edge_mlp(
            edges, recv, send,
            ew0[:D], ew0[D:2 * D], ew0[2 * D:],
            b2d(params["edge_b0"][i]), params["edge_W1"][i],
            b2d(params["edge_b1"][i]), params["edge_W2"][i],
            b2d(params["edge_b2"][i]), b2d(params["edge_ln_g"][i]),
            b2d(params["edge_ln_b"][i]))
        parts = scatter(edges, idx0_s, zeros)
        nw0 = params["node_W0"][i]
        nodes = node_mlp(
            parts[0, :N], parts[1, :N], nodes,
            nw0[:D], nw0[D:],
            b2d(params["node_b0"][i]), params["node_W1"][i],
            b2d(params["node_b1"][i]), params["node_W2"][i],
            b2d(params["node_b2"][i]), b2d(params["node_ln_g"][i]),
            b2d(params["node_ln_b"][i]))

    return nodes, edges, neighbor_idxs


# speedup vs baseline: 3.6028x; 1.1382x over previous
import functools

import jax
import jax.numpy as jnp
from jax import lax
from jax.experimental import pallas as pl
from jax.experimental.pallas import tpu as pltpu
from jax.experimental.pallas import tpu_sc as plsc

_NC = 2
_NS = 16
_NW = _NC * _NS

_NB = 5

_F32 = jnp.float32


def _gather_kernel(E, D, gb):
    ew = E // _NW
    gc = ew // gb
    assert gc % _NB == 0 and gc >= 2 * _NB and gb % 8 == 0 and gb <= 128
    nt2 = 2 * gc
    mesh = plsc.VectorSubcoreMesh(core_axis_name="c", subcore_axis_name="s")

    @functools.partial(
        pl.kernel,
        out_type=[jax.ShapeDtypeStruct((E, D), _F32),
                  jax.ShapeDtypeStruct((E, D), _F32)],
        mesh=mesh,
        scratch_types=[
            pltpu.VMEM((gc, gb), jnp.int32),
            pltpu.VMEM((gc, gb), jnp.int32),
        ] + [pltpu.VMEM((gb, D), _F32)] * _NB
          + [pltpu.SemaphoreType.DMA] * (2 * _NB),
    )
    def gather(nodes_h, idx0_h, idx1_h, recv_h, send_h, idx0_v, idx1_v, *rest):
        bufs = rest[0:_NB]
        gsem = rest[_NB:2 * _NB]
        wsem = rest[2 * _NB:3 * _NB]
        c = lax.axis_index("c")
        s = lax.axis_index("s")
        w = s * _NC + c
        base_e = w * ew
        pltpu.sync_copy(idx0_h.at[w], idx0_v)
        pltpu.sync_copy(idx1_h.at[w], idx1_v)

        def fire(ch, parity, slot):
            idx = idx0_v if parity == 0 else idx1_v
            pltpu.async_copy(nodes_h.at[idx.at[ch]], bufs[slot], gsem[slot])

        fire(0, 0, 0)
        fire(0, 1, 1)

        def body(i, carry):
            for p in range(2 * _NB):
                ch = i * _NB + p // 2
                sl = p % _NB
                sp = (p + 2) % _NB
                t = i * 2 * _NB + p

                def drain():
                    pltpu.make_async_copy(
                        bufs[sp], recv_h.at[pl.ds(base_e, gb), :],
                        wsem[sp]).wait()

                if p >= 3:
                    drain()
                else:
                    pl.when(t >= 3)(drain)

                @pl.when(t + 2 < nt2)
                def _():
                    fire(ch + 1, p % 2, sp)

                pltpu.make_async_copy(
                    recv_h.at[pl.ds(base_e, gb), :], bufs[sl],
                    gsem[sl]).wait()
                off = base_e + ch * gb
                dst = recv_h if p % 2 == 0 else send_h
                pltpu.async_copy(bufs[sl], dst.at[pl.ds(off, gb), :],
                                 wsem[sl])
            return carry

        lax.fori_loop(0, gc // _NB, body, 0)
        for t in (nt2 - 3, nt2 - 2, nt2 - 1):
            sp = t % _NB
            pltpu.make_async_copy(
                bufs[sp], recv_h.at[pl.ds(base_e, gb), :], wsem[sp]).wait()

    return gather


def _scatter_kernel(E, NP, D, gbs):
    ew = E // _NW
    gc = ew // gbs
    nt = NP // _NS
    assert gc % _NB == 0 and gbs % 8 == 0 and gbs <= 128
    mesh = plsc.VectorSubcoreMesh(core_axis_name="c", subcore_axis_name="s")

    @functools.partial(
        pl.kernel,
        out_type=jax.ShapeDtypeStruct((_NC, NP, D), _F32),
        mesh=mesh,
        scratch_types=[
            pltpu.VMEM((gc, gbs), jnp.int32),
            pltpu.VMEM_SHARED((NP, D), _F32),
        ] + [pltpu.VMEM((gbs, D), _F32)] * _NB
          + [pltpu.SemaphoreType.DMA] * _NB,
    )
    def scatter(edges_h, idx0_h, zeros_h, out_h, idx_v, agg_sh, *rest):
        bufs = rest[0:_NB]
        rsem = rest[_NB:2 * _NB]
        c = lax.axis_index("c")
        s = lax.axis_index("s")
        w = s * _NC + c
        base_e = w * ew
        pltpu.sync_copy(zeros_h.at[pl.ds(s * nt, nt), :],
                        agg_sh.at[pl.ds(s * nt, nt), :])
        pltpu.sync_copy(idx0_h.at[w], idx_v)
        plsc.subcore_barrier()

        pltpu.async_copy(edges_h.at[pl.ds(base_e, gbs), :], bufs[0], rsem[0])

        def body(i, carry):
            for p in range(_NB):
                ch = i * _NB + p
                sp = (p + 1) % _NB
                pltpu.make_async_copy(
                    edges_h.at[pl.ds(base_e, gbs), :], bufs[p],
                    rsem[p]).wait()

                @pl.when(ch + 1 < gc)
                def _():
                    pltpu.async_copy(
                        edges_h.at[pl.ds(base_e + (ch + 1) * gbs, gbs), :],
                        bufs[sp], rsem[sp])

                pltpu.sync_copy(bufs[p], agg_sh.at[idx_v.at[ch]], add=True)
            return carry

        lax.fori_loop(0, gc // _NB, body, 0)
        plsc.subcore_barrier()
        pltpu.sync_copy(agg_sh.at[pl.ds(s * nt, nt), :],
                        out_h.at[c, pl.ds(s * nt, nt), :])

    return scatter


def _edge_mlp_body(ed, rv, sd, w0e, w0r, w0s, b0, w1, b1, w2, b2, g, bn, out):
    x = jnp.dot(ed[...], w0e[...], preferred_element_type=_F32)
    x += jnp.dot(rv[...], w0r[...], preferred_element_type=_F32)
    x += jnp.dot(sd[...], w0s[...], preferred_element_type=_F32)
    x = jax.nn.relu(x + b0[...])
    x = jax.nn.relu(jnp.dot(x, w1[...], preferred_element_type=_F32) + b1[...])
    y = jnp.dot(x, w2[...], preferred_element_type=_F32) + b2[...]
    mu = jnp.mean(y, axis=-1, keepdims=True)
    var = jnp.mean((y - mu) ** 2, axis=-1, keepdims=True)
    out[...] = (y - mu) / jnp.sqrt(var + 1e-5) * g[...] + bn[...]


def _node_mlp_body(pa, pb, pc, pd, nd, w0a, w0n, b0, w1, b1, w2, b2, g, bn,
                   out):
    agg = (pa[...] + pb[...]) + (pc[...] + pd[...])
    x = jnp.dot(agg, w0a[...], preferred_element_type=_F32)
    x += jnp.dot(nd[...], w0n[...], preferred_element_type=_F32)
    x = jax.nn.relu(x + b0[...])
    x = jax.nn.relu(jnp.dot(x, w1[...], preferred_element_type=_F32) + b1[...])
    y = jnp.dot(x, w2[...], preferred_element_type=_F32) + b2[...]
    mu = jnp.mean(y, axis=-1, keepdims=True)
    var = jnp.mean((y - mu) ** 2, axis=-1, keepdims=True)
    out[...] = (y - mu) / jnp.sqrt(var + 1e-5) * g[...] + bn[...]


def _row_spec(tile, d):
    return pl.BlockSpec((tile, d), lambda i: (i, 0))


def _full_spec(shape):
    return pl.BlockSpec(shape, lambda i: (0,) * len(shape))


def _edge_mlp_call(E, D, H, tile):
    grid = (E // tile,)
    w = _full_spec((D, H))
    wh = _full_spec((H, H))
    w2 = _full_spec((H, D))
    b = _full_spec((1, H))
    bd = _full_spec((1, D))
    return pl.pallas_call(
        _edge_mlp_body,
        grid=grid,
        in_specs=[_row_spec(tile, D)] * 3 + [w, w, w, b, wh, b, w2, bd, bd, bd],
        out_specs=_row_spec(tile, D),
        out_shape=jax.ShapeDtypeStruct((E, D), _F32),
    )


def _node_mlp_call(N, D, H, tile):
    grid = (N // tile,)
    w = _full_spec((D, H))
    wh = _full_spec((H, H))
    w2 = _full_spec((H, D))
    b = _full_spec((1, H))
    bd = _full_spec((1, D))
    return pl.pallas_call(
        _node_mlp_body,
        grid=grid,
        in_specs=[_row_spec(tile, D)] * 5 + [w, w, b, wh, b, w2, bd, bd, bd],
        out_specs=_row_spec(tile, D),
        out_shape=jax.ShapeDtypeStruct((N, D), _F32),
    )


def kernel(nodes, edges, neighbor_idxs, params):
    N, D = nodes.shape
    E = edges.shape[0]
    L = params["edge_W0"].shape[0]
    H = params["edge_W1"].shape[1]

    E2 = E // 2
    GB = 40
    ew = E2 // _NW
    gc = ew // GB
    idx0 = neighbor_idxs[:, 0].astype(jnp.int32)
    idx1 = neighbor_idxs[:, 1].astype(jnp.int32)
    idx0_h = [idx0[h * E2:(h + 1) * E2].reshape(_NW, gc, GB) for h in range(2)]
    idx1_h = [idx1[h * E2:(h + 1) * E2].reshape(_NW, gc, GB) for h in range(2)]
    NP = ((N + 8 * _NS - 1) // (8 * _NS)) * (8 * _NS)
    zeros = jnp.zeros((NP, D), _F32)

    gather = _gather_kernel(E2, D, GB)
    scatter = _scatter_kernel(E2, NP, D, GB)
    edge_mlp = _edge_mlp_call(E2, D, H, tile=2000)
    node_mlp = _node_mlp_call(N, D, H, tile=2000)

    eh = [edges[:E2], edges[E2:]]

    def b2d(b):
        return b.reshape(1, -1)

    for i in range(L):
        ew0 = params["edge_W0"][i]
        edge_args = (
            ew0[:D], ew0[D:2 * D], ew0[2 * D:],
            b2d(params["edge_b0"][i]), params["edge_W1"][i],
            b2d(params["edge_b1"][i]), params["edge_W2"][i],
            b2d(params["edge_b2"][i]), b2d(params["edge_ln_g"][i]),
            b2d(params["edge_ln_b"][i]))
        parts = []
        for h in range(2):
            recv, send = gather(nodes, idx0_h[h], idx1_h[h])
            eh[h] = edge_mlp(eh[h], recv, send, *edge_args)
            parts.append(scatter(eh[h], idx0_h[h], zeros))
        nw0 = params["node_W0"][i]
        nodes = node_mlp(
            parts[0][0, :N], parts[0][1, :N],
            parts[1][0, :N], parts[1][1, :N], nodes,
            nw0[:D], nw0[D:],
            b2d(params["node_b0"][i]), params["node_W1"][i],
            b2d(params["node_b1"][i]), params["node_W2"][i],
            b2d(params["node_b2"][i]), b2d(params["node_ln_g"][i]),
            b2d(params["node_ln_b"][i]))

    return nodes, jnp.concatenate(eh, axis=0), neighbor_idxs


# speedup vs baseline: 3.6307x; 1.0077x over previous
import functools

import jax
import jax.numpy as jnp
from jax import lax
from jax.experimental import pallas as pl
from jax.experimental.pallas import tpu as pltpu
from jax.experimental.pallas import tpu_sc as plsc

_NC = 2
_NS = 16
_NW = _NC * _NS

_NB = 5

_F32 = jnp.float32


def _gather_kernel(E, D, gb):
    ew = E // _NW
    gc = ew // gb
    assert gc % _NB == 0 and gc >= 2 * _NB and gb % 8 == 0 and gb <= 128
    nt2 = 2 * gc
    mesh = plsc.VectorSubcoreMesh(core_axis_name="c", subcore_axis_name="s")

    @functools.partial(
        pl.kernel,
        out_type=[jax.ShapeDtypeStruct((E, D), _F32),
                  jax.ShapeDtypeStruct((E, D), _F32)],
        mesh=mesh,
        scratch_types=[
            pltpu.VMEM((gc, gb), jnp.int32),
            pltpu.VMEM((gc, gb), jnp.int32),
        ] + [pltpu.VMEM((gb, D), _F32)] * _NB
          + [pltpu.SemaphoreType.DMA] * (2 * _NB),
    )
    def gather(nodes_h, idx0_h, idx1_h, recv_h, send_h, idx0_v, idx1_v, *rest):
        bufs = rest[0:_NB]
        gsem = rest[_NB:2 * _NB]
        wsem = rest[2 * _NB:3 * _NB]
        c = lax.axis_index("c")
        s = lax.axis_index("s")
        w = s * _NC + c
        base_e = w * ew
        pltpu.sync_copy(idx0_h.at[w], idx0_v)
        pltpu.sync_copy(idx1_h.at[w], idx1_v)

        def fire(ch, parity, slot):
            idx = idx0_v if parity == 0 else idx1_v
            pltpu.async_copy(nodes_h.at[idx.at[ch]], bufs[slot], gsem[slot])

        fire(0, 0, 0)
        fire(0, 1, 1)

        def body(i, carry):
            for p in range(2 * _NB):
                ch = i * _NB + p // 2
                sl = p % _NB
                sp = (p + 2) % _NB
                t = i * 2 * _NB + p

                def drain():
                    pltpu.make_async_copy(
                        bufs[sp], recv_h.at[pl.ds(base_e, gb), :],
                        wsem[sp]).wait()

                if p >= 3:
                    drain()
                else:
                    pl.when(t >= 3)(drain)

                @pl.when(t + 2 < nt2)
                def _():
                    fire(ch + 1, p % 2, sp)

                pltpu.make_async_copy(
                    recv_h.at[pl.ds(base_e, gb), :], bufs[sl],
                    gsem[sl]).wait()
                off = base_e + ch * gb
                dst = recv_h if p % 2 == 0 else send_h
                pltpu.async_copy(bufs[sl], dst.at[pl.ds(off, gb), :],
                                 wsem[sl])
            return carry

        lax.fori_loop(0, gc // _NB, body, 0)
        for t in (nt2 - 3, nt2 - 2, nt2 - 1):
            sp = t % _NB
            pltpu.make_async_copy(
                bufs[sp], recv_h.at[pl.ds(base_e, gb), :], wsem[sp]).wait()

    return gather


def _scatter_kernel(E, NP, D, gbs):
    ew = E // _NW
    gc = ew // gbs
    nt = NP // _NS
    assert gc % _NB == 0 and gbs % 8 == 0 and gbs <= 128
    mesh = plsc.VectorSubcoreMesh(core_axis_name="c", subcore_axis_name="s")

    @functools.partial(
        pl.kernel,
        out_type=jax.ShapeDtypeStruct((_NC, NP, D), _F32),
        mesh=mesh,
        scratch_types=[
            pltpu.VMEM((gc, gbs), jnp.int32),
            pltpu.VMEM_SHARED((NP, D), _F32),
        ] + [pltpu.VMEM((gbs, D), _F32)] * _NB
          + [pltpu.SemaphoreType.DMA] * _NB,
    )
    def scatter(edges_h, idx0_h, init_h, out_h, idx_v, agg_sh, *rest):
        bufs = rest[0:_NB]
        rsem = rest[_NB:2 * _NB]
        c = lax.axis_index("c")
        s = lax.axis_index("s")
        w = s * _NC + c
        base_e = w * ew
        pltpu.sync_copy(init_h.at[c, pl.ds(s * nt, nt), :],
                        agg_sh.at[pl.ds(s * nt, nt), :])
        pltpu.sync_copy(idx0_h.at[w], idx_v)
        plsc.subcore_barrier()

        pltpu.async_copy(edges_h.at[pl.ds(base_e, gbs), :], bufs[0], rsem[0])

        def body(i, carry):
            for p in range(_NB):
                ch = i * _NB + p
                sp = (p + 1) % _NB
                pltpu.make_async_copy(
                    edges_h.at[pl.ds(base_e, gbs), :], bufs[p],
                    rsem[p]).wait()

                @pl.when(ch + 1 < gc)
                def _():
                    pltpu.async_copy(
                        edges_h.at[pl.ds(base_e + (ch + 1) * gbs, gbs), :],
                        bufs[sp], rsem[sp])

                pltpu.sync_copy(bufs[p], agg_sh.at[idx_v.at[ch]], add=True)
            return carry

        lax.fori_loop(0, gc // _NB, body, 0)
        plsc.subcore_barrier()
        pltpu.sync_copy(agg_sh.at[pl.ds(s * nt, nt), :],
                        out_h.at[c, pl.ds(s * nt, nt), :])

    return scatter


def _edge_mlp_body(ed, rv, sd, w0e, w0r, w0s, b0, w1, b1, w2, b2, g, bn, out):
    x = jnp.dot(ed[...], w0e[...], preferred_element_type=_F32)
    x += jnp.dot(rv[...], w0r[...], preferred_element_type=_F32)
    x += jnp.dot(sd[...], w0s[...], preferred_element_type=_F32)
    x = jax.nn.relu(x + b0[...])
    x = jax.nn.relu(jnp.dot(x, w1[...], preferred_element_type=_F32) + b1[...])
    y = jnp.dot(x, w2[...], preferred_element_type=_F32) + b2[...]
    mu = jnp.mean(y, axis=-1, keepdims=True)
    var = jnp.mean((y - mu) ** 2, axis=-1, keepdims=True)
    out[...] = (y - mu) / jnp.sqrt(var + 1e-5) * g[...] + bn[...]


def _node_mlp_body(pa, pb, nd, w0a, w0n, b0, w1, b1, w2, b2, g, bn, out):
    agg = pa[...] + pb[...]
    x = jnp.dot(agg, w0a[...], preferred_element_type=_F32)
    x += jnp.dot(nd[...], w0n[...], preferred_element_type=_F32)
    x = jax.nn.relu(x + b0[...])
    x = jax.nn.relu(jnp.dot(x, w1[...], preferred_element_type=_F32) + b1[...])
    y = jnp.dot(x, w2[...], preferred_element_type=_F32) + b2[...]
    mu = jnp.mean(y, axis=-1, keepdims=True)
    var = jnp.mean((y - mu) ** 2, axis=-1, keepdims=True)
    out[...] = (y - mu) / jnp.sqrt(var + 1e-5) * g[...] + bn[...]


def _row_spec(tile, d):
    return pl.BlockSpec((tile, d), lambda i: (i, 0))


def _full_spec(shape):
    return pl.BlockSpec(shape, lambda i: (0,) * len(shape))


def _edge_mlp_call(E, D, H, tile):
    grid = (E // tile,)
    w = _full_spec((D, H))
    wh = _full_spec((H, H))
    w2 = _full_spec((H, D))
    b = _full_spec((1, H))
    bd = _full_spec((1, D))
    return pl.pallas_call(
        _edge_mlp_body,
        grid=grid,
        in_specs=[_row_spec(tile, D)] * 3 + [w, w, w, b, wh, b, w2, bd, bd, bd],
        out_specs=_row_spec(tile, D),
        out_shape=jax.ShapeDtypeStruct((E, D), _F32),
    )


def _node_mlp_call(N, D, H, tile):
    grid = (N // tile,)
    w = _full_spec((D, H))
    wh = _full_spec((H, H))
    w2 = _full_spec((H, D))
    b = _full_spec((1, H))
    bd = _full_spec((1, D))
    return pl.pallas_call(
        _node_mlp_body,
        grid=grid,
        in_specs=[_row_spec(tile, D)] * 3 + [w, w, b, wh, b, w2, bd, bd, bd],
        out_specs=_row_spec(tile, D),
        out_shape=jax.ShapeDtypeStruct((N, D), _F32),
    )


def kernel(nodes, edges, neighbor_idxs, params):
    N, D = nodes.shape
    E = edges.shape[0]
    L = params["edge_W0"].shape[0]
    H = params["edge_W1"].shape[1]

    E2 = E // 2
    GB = 40
    ew = E2 // _NW
    gc = ew // GB
    idx0 = neighbor_idxs[:, 0].astype(jnp.int32)
    idx1 = neighbor_idxs[:, 1].astype(jnp.int32)
    idx0_h = [idx0[h * E2:(h + 1) * E2].reshape(_NW, gc, GB) for h in range(2)]
    idx1_h = [idx1[h * E2:(h + 1) * E2].reshape(_NW, gc, GB) for h in range(2)]
    NP = ((N + 8 * _NS - 1) // (8 * _NS)) * (8 * _NS)
    zeros = jnp.zeros((_NC, NP, D), _F32)

    gather = _gather_kernel(E2, D, GB)
    scatter = _scatter_kernel(E2, NP, D, GB)
    edge_mlp = _edge_mlp_call(E2, D, H, tile=2000)
    node_mlp = _node_mlp_call(N, D, H, tile=2000)

    eh = [edges[:E2], edges[E2:]]

    def b2d(b):
        return b.reshape(1, -1)

    for i in range(L):
        ew0 = params["edge_W0"][i]
        edge_args = (
            ew0[:D], ew0[D:2 * D], ew0[2 * D:],
            b2d(params["edge_b0"][i]), params["edge_W1"][i],
            b2d(params["edge_b1"][i]), params["edge_W2"][i],
            b2d(params["edge_b2"][i]), b2d(params["edge_ln_g"][i]),
            b2d(params["edge_ln_b"][i]))
        parts = zeros
        for h in range(2):
            recv, send = gather(nodes, idx0_h[h], idx1_h[h])
            eh[h] = edge_mlp(eh[h], recv, send, *edge_args)
            parts = scatter(eh[h], idx0_h[h], parts)
        nw0 = params["node_W0"][i]
        nodes = node_mlp(
            parts[0, :N], parts[1, :N], nodes,
            nw0[:D], nw0[D:],
            b2d(params["node_b0"][i]), params["node_W1"][i],
            b2d(params["node_b1"][i]), params["node_W2"][i],
            b2d(params["node_b2"][i]), b2d(params["node_ln_g"][i]),
            b2d(params["node_ln_b"][i]))

    return nodes, jnp.concatenate(eh, axis=0), neighbor_idxs


# speedup vs baseline: 3.6770x; 1.0128x over previous
import functools

import jax
import jax.numpy as jnp
from jax import lax
from jax.experimental import pallas as pl
from jax.experimental.pallas import tpu as pltpu
from jax.experimental.pallas import tpu_sc as plsc

_NC = 2
_NS = 16
_NW = _NC * _NS

_NB = 5

_F32 = jnp.float32


def _gather_kernel(E, D, gb):
    ew = E // _NW
    gc = ew // gb
    assert gc % _NB == 0 and gc >= 2 * _NB and gb % 8 == 0 and gb <= 128
    nt2 = 2 * gc
    mesh = plsc.VectorSubcoreMesh(core_axis_name="c", subcore_axis_name="s")

    @functools.partial(
        pl.kernel,
        out_type=[jax.ShapeDtypeStruct((E, D), _F32),
                  jax.ShapeDtypeStruct((E, D), _F32)],
        mesh=mesh,
        scratch_types=[
            pltpu.VMEM((gc, gb), jnp.int32),
            pltpu.VMEM((gc, gb), jnp.int32),
        ] + [pltpu.VMEM((gb, D), _F32)] * _NB
          + [pltpu.SemaphoreType.DMA] * (2 * _NB),
    )
    def gather(nodes_h, idx0_h, idx1_h, recv_h, send_h, idx0_v, idx1_v, *rest):
        bufs = rest[0:_NB]
        gsem = rest[_NB:2 * _NB]
        wsem = rest[2 * _NB:3 * _NB]
        c = lax.axis_index("c")
        s = lax.axis_index("s")
        w = s * _NC + c
        base_e = w * ew
        pltpu.sync_copy(idx0_h.at[w], idx0_v)
        pltpu.sync_copy(idx1_h.at[w], idx1_v)

        def fire(ch, parity, slot):
            idx = idx0_v if parity == 0 else idx1_v
            pltpu.async_copy(nodes_h.at[idx.at[ch]], bufs[slot], gsem[slot])

        fire(0, 0, 0)
        fire(0, 1, 1)

        def body(i, carry):
            for p in range(2 * _NB):
                ch = i * _NB + p // 2
                sl = p % _NB
                sp = (p + 2) % _NB
                t = i * 2 * _NB + p

                def drain():
                    pltpu.make_async_copy(
                        bufs[sp], recv_h.at[pl.ds(base_e, gb), :],
                        wsem[sp]).wait()

                if p >= 3:
                    drain()
                else:
                    pl.when(t >= 3)(drain)

                @pl.when(t + 2 < nt2)
                def _():
                    fire(ch + 1, p % 2, sp)

                pltpu.make_async_copy(
                    recv_h.at[pl.ds(base_e, gb), :], bufs[sl],
                    gsem[sl]).wait()
                off = base_e + ch * gb
                dst = recv_h if p % 2 == 0 else send_h
                pltpu.async_copy(bufs[sl], dst.at[pl.ds(off, gb), :],
                                 wsem[sl])
            return carry

        lax.fori_loop(0, gc // _NB, body, 0)
        for t in (nt2 - 3, nt2 - 2, nt2 - 1):
            sp = t % _NB
            pltpu.make_async_copy(
                bufs[sp], recv_h.at[pl.ds(base_e, gb), :], wsem[sp]).wait()

    return gather


def _scatter_kernel(E, NP, D, gbs):
    ew = E // _NW
    gc = ew // gbs
    nt = NP // _NS
    assert gc % _NB == 0 and gbs % 8 == 0 and gbs <= 128
    mesh = plsc.VectorSubcoreMesh(core_axis_name="c", subcore_axis_name="s")

    @functools.partial(
        pl.kernel,
        out_type=jax.ShapeDtypeStruct((_NC, NP, D), _F32),
        mesh=mesh,
        scratch_types=[
            pltpu.VMEM((gc, gbs), jnp.int32),
            pltpu.VMEM_SHARED((NP, D), _F32),
        ] + [pltpu.VMEM((gbs, D), _F32)] * _NB
          + [pltpu.SemaphoreType.DMA] * _NB,
    )
    def scatter(edges_h, idx0_h, init_h, out_h, idx_v, agg_sh, *rest):
        bufs = rest[0:_NB]
        rsem = rest[_NB:2 * _NB]
        c = lax.axis_index("c")
        s = lax.axis_index("s")
        w = s * _NC + c
        base_e = w * ew
        pltpu.sync_copy(init_h.at[c, pl.ds(s * nt, nt), :],
                        agg_sh.at[pl.ds(s * nt, nt), :])
        pltpu.sync_copy(idx0_h.at[w], idx_v)
        plsc.subcore_barrier()

        pltpu.async_copy(edges_h.at[pl.ds(base_e, gbs), :], bufs[0], rsem[0])

        def body(i, carry):
            for p in range(_NB):
                ch = i * _NB + p
                sp = (p + 1) % _NB
                pltpu.make_async_copy(
                    edges_h.at[pl.ds(base_e, gbs), :], bufs[p],
                    rsem[p]).wait()

                @pl.when(ch + 1 < gc)
                def _():
                    pltpu.async_copy(
                        edges_h.at[pl.ds(base_e + (ch + 1) * gbs, gbs), :],
                        bufs[sp], rsem[sp])

                pltpu.sync_copy(bufs[p], agg_sh.at[idx_v.at[ch]], add=True)
            return carry

        lax.fori_loop(0, gc // _NB, body, 0)
        plsc.subcore_barrier()
        pltpu.sync_copy(agg_sh.at[pl.ds(s * nt, nt), :],
                        out_h.at[c, pl.ds(s * nt, nt), :])

    return scatter


def _edge_mlp_body(ed, rv, sd, w0e, w0r, w0s, b0, w1, b1, w2, b2, g, bn, out):
    x = jnp.dot(ed[...], w0e[...], preferred_element_type=_F32)
    x += jnp.dot(rv[...], w0r[...], preferred_element_type=_F32)
    x += jnp.dot(sd[...], w0s[...], preferred_element_type=_F32)
    x = jax.nn.relu(x + b0[...])
    x = jax.nn.relu(jnp.dot(x, w1[...], preferred_element_type=_F32) + b1[...])
    y = jnp.dot(x, w2[...], preferred_element_type=_F32) + b2[...]
    mu = jnp.mean(y, axis=-1, keepdims=True)
    var = jnp.mean((y - mu) ** 2, axis=-1, keepdims=True)
    out[...] = (y - mu) / jnp.sqrt(var + 1e-5) * g[...] + bn[...]


def _node_mlp_body(pa, pb, nd, w0a, w0n, b0, w1, b1, w2, b2, g, bn, out):
    agg = pa[...] + pb[...]
    x = jnp.dot(agg, w0a[...], preferred_element_type=_F32)
    x += jnp.dot(nd[...], w0n[...], preferred_element_type=_F32)
    x = jax.nn.relu(x + b0[...])
    x = jax.nn.relu(jnp.dot(x, w1[...], preferred_element_type=_F32) + b1[...])
    y = jnp.dot(x, w2[...], preferred_element_type=_F32) + b2[...]
    mu = jnp.mean(y, axis=-1, keepdims=True)
    var = jnp.mean((y - mu) ** 2, axis=-1, keepdims=True)
    out[...] = (y - mu) / jnp.sqrt(var + 1e-5) * g[...] + bn[...]


def _row_spec(tile, d):
    return pl.BlockSpec((tile, d), lambda i: (i, 0))


def _full_spec(shape):
    return pl.BlockSpec(shape, lambda i: (0,) * len(shape))


def _edge_mlp_call(E, D, H, tile, ed_off=0):
    grid = (E // tile,)
    w = _full_spec((D, H))
    wh = _full_spec((H, H))
    w2 = _full_spec((H, D))
    b = _full_spec((1, H))
    bd = _full_spec((1, D))
    ed_spec = pl.BlockSpec((tile, D), lambda i: (i + ed_off, 0))
    return pl.pallas_call(
        _edge_mlp_body,
        grid=grid,
        in_specs=[ed_spec] + [_row_spec(tile, D)] * 2
                 + [w, w, w, b, wh, b, w2, bd, bd, bd],
        out_specs=_row_spec(tile, D),
        out_shape=jax.ShapeDtypeStruct((E, D), _F32),
    )


def _node_mlp_call(N, D, H, tile):
    grid = (N // tile,)
    w = _full_spec((D, H))
    wh = _full_spec((H, H))
    w2 = _full_spec((H, D))
    b = _full_spec((1, H))
    bd = _full_spec((1, D))
    return pl.pallas_call(
        _node_mlp_body,
        grid=grid,
        in_specs=[_row_spec(tile, D)] * 3 + [w, w, b, wh, b, w2, bd, bd, bd],
        out_specs=_row_spec(tile, D),
        out_shape=jax.ShapeDtypeStruct((N, D), _F32),
    )


def kernel(nodes, edges, neighbor_idxs, params):
    N, D = nodes.shape
    E = edges.shape[0]
    L = params["edge_W0"].shape[0]
    H = params["edge_W1"].shape[1]

    E2 = E // 2
    GB = 40
    ew = E2 // _NW
    gc = ew // GB
    idx0 = neighbor_idxs[:, 0].astype(jnp.int32)
    idx1 = neighbor_idxs[:, 1].astype(jnp.int32)
    idx0_h = [idx0[h * E2:(h + 1) * E2].reshape(_NW, gc, GB) for h in range(2)]
    idx1_h = [idx1[h * E2:(h + 1) * E2].reshape(_NW, gc, GB) for h in range(2)]
    NP = ((N + 8 * _NS - 1) // (8 * _NS)) * (8 * _NS)
    zeros = jnp.zeros((_NC, NP, D), _F32)

    TILE = 2000
    gather = _gather_kernel(E2, D, GB)
    scatter = _scatter_kernel(E2, NP, D, GB)
    edge_mlp = _edge_mlp_call(E2, D, H, tile=TILE)
    edge_mlp_l0 = [_edge_mlp_call(E2, D, H, tile=TILE,
                                  ed_off=h * (E2 // TILE)) for h in range(2)]
    node_mlp = _node_mlp_call(N, D, H, tile=TILE)

    eh = [edges, edges]

    def b2d(b):
        return b.reshape(1, -1)

    for i in range(L):
        ew0 = params["edge_W0"][i]
        edge_args = (
            ew0[:D], ew0[D:2 * D], ew0[2 * D:],
            b2d(params["edge_b0"][i]), params["edge_W1"][i],
            b2d(params["edge_b1"][i]), params["edge_W2"][i],
            b2d(params["edge_b2"][i]), b2d(params["edge_ln_g"][i]),
            b2d(params["edge_ln_b"][i]))
        parts = zeros
        for h in range(2):
            recv, send = gather(nodes, idx0_h[h], idx1_h[h])
            mlp = edge_mlp_l0[h] if i == 0 else edge_mlp
            eh[h] = mlp(eh[h], recv, send, *edge_args)
            parts = scatter(eh[h], idx0_h[h], parts)
        nw0 = params["node_W0"][i]
        nodes = node_mlp(
            parts[0, :N], parts[1, :N], nodes,
            nw0[:D], nw0[D:],
            b2d(params["node_b0"][i]), params["node_W1"][i],
            b2d(params["node_b1"][i]), params["node_W2"][i],
            b2d(params["node_b2"][i]), b2d(params["node_ln_g"][i]),
            b2d(params["node_ln_b"][i]))

    return nodes, jnp.concatenate(eh, axis=0), neighbor_idxs


# speedup vs baseline: 4.3280x; 1.1771x over previous
import functools

import jax
import jax.numpy as jnp
from jax import lax
from jax.experimental import pallas as pl
from jax.experimental.pallas import tpu as pltpu
from jax.experimental.pallas import tpu_sc as plsc

_NC = 2
_NS = 16
_NW = _NC * _NS

_NB = 5

_F32 = jnp.float32


def _gather_kernel(E, D, gb):
    ew = E // _NW
    gc = ew // gb
    assert gc % _NB == 0 and gc >= 2 * _NB and gb % 8 == 0 and gb <= 128
    nt2 = 2 * gc
    mesh = plsc.VectorSubcoreMesh(core_axis_name="c", subcore_axis_name="s")

    @functools.partial(
        pl.kernel,
        out_type=[jax.ShapeDtypeStruct((E, D), _F32),
                  jax.ShapeDtypeStruct((E, D), _F32)],
        mesh=mesh,
        scratch_types=[
            pltpu.VMEM((gc, gb), jnp.int32),
            pltpu.VMEM((gc, gb), jnp.int32),
        ] + [pltpu.VMEM((gb, D), _F32)] * _NB
          + [pltpu.SemaphoreType.DMA] * (2 * _NB),
    )
    def gather(nodes_h, idx0_h, idx1_h, recv_h, send_h, idx0_v, idx1_v, *rest):
        bufs = rest[0:_NB]
        gsem = rest[_NB:2 * _NB]
        wsem = rest[2 * _NB:3 * _NB]
        c = lax.axis_index("c")
        s = lax.axis_index("s")
        w = s * _NC + c
        base_e = w * ew
        pltpu.sync_copy(idx0_h.at[w], idx0_v)
        pltpu.sync_copy(idx1_h.at[w], idx1_v)

        def fire(ch, parity, slot):
            idx = idx0_v if parity == 0 else idx1_v
            pltpu.async_copy(nodes_h.at[idx.at[ch]], bufs[slot], gsem[slot])

        fire(0, 0, 0)
        fire(0, 1, 1)

        def body(i, carry):
            for p in range(2 * _NB):
                ch = i * _NB + p // 2
                sl = p % _NB
                sp = (p + 2) % _NB
                t = i * 2 * _NB + p

                def drain():
                    pltpu.make_async_copy(
                        bufs[sp], recv_h.at[pl.ds(base_e, gb), :],
                        wsem[sp]).wait()

                if p >= 3:
                    drain()
                else:
                    pl.when(t >= 3)(drain)

                @pl.when(t + 2 < nt2)
                def _():
                    fire(ch + 1, p % 2, sp)

                pltpu.make_async_copy(
                    recv_h.at[pl.ds(base_e, gb), :], bufs[sl],
                    gsem[sl]).wait()
                off = base_e + ch * gb
                dst = recv_h if p % 2 == 0 else send_h
                pltpu.async_copy(bufs[sl], dst.at[pl.ds(off, gb), :],
                                 wsem[sl])
            return carry

        lax.fori_loop(0, gc // _NB, body, 0)
        for t in (nt2 - 3, nt2 - 2, nt2 - 1):
            sp = t % _NB
            pltpu.make_async_copy(
                bufs[sp], recv_h.at[pl.ds(base_e, gb), :], wsem[sp]).wait()

    return gather


def _gather_add_kernel(E, D, gb):
    ew = E // _NW
    gc = ew // gb
    assert gc % _NB == 0 and gc >= 2 * _NB and gb % 8 == 0 and gb <= 128
    mesh = plsc.VectorSubcoreMesh(core_axis_name="c", subcore_axis_name="s")

    @functools.partial(
        pl.kernel,
        out_type=jax.ShapeDtypeStruct((E, D), _F32),
        mesh=mesh,
        scratch_types=[
            pltpu.VMEM((gc, gb), jnp.int32),
            pltpu.VMEM((gc, gb), jnp.int32),
        ] + [pltpu.VMEM((gb, D), _F32)] * (2 * _NB)
          + [pltpu.SemaphoreType.DMA] * (3 * _NB),
    )
    def gather_add(p0_h, p1_h, idx0_h, idx1_h, pre_h, idx0_v, idx1_v, *rest):
        bufs_a = rest[0:_NB]
        bufs_b = rest[_NB:2 * _NB]
        gsem_a = rest[2 * _NB:3 * _NB]
        gsem_b = rest[3 * _NB:4 * _NB]
        wsem = rest[4 * _NB:5 * _NB]
        c = lax.axis_index("c")
        s = lax.axis_index("s")
        w = s * _NC + c
        base_e = w * ew
        pltpu.sync_copy(idx0_h.at[w], idx0_v)
        pltpu.sync_copy(idx1_h.at[w], idx1_v)

        def fire(ch, slot):
            pltpu.async_copy(p0_h.at[idx0_v.at[ch]], bufs_a[slot],
                             gsem_a[slot])
            pltpu.async_copy(p1_h.at[idx1_v.at[ch]], bufs_b[slot],
                             gsem_b[slot])

        fire(0, 0)
        fire(1, 1)

        def body(i, carry):
            for p in range(_NB):
                ch = i * _NB + p
                sp = (p + 2) % _NB

                def drain():
                    pltpu.make_async_copy(
                        bufs_a[sp], pre_h.at[pl.ds(base_e, gb), :],
                        wsem[sp]).wait()

                if p >= 3:
                    drain()
                else:
                    pl.when(ch >= 3)(drain)

                @pl.when(ch + 2 < gc)
                def _():
                    fire(ch + 2, sp)

                pltpu.make_async_copy(
                    pre_h.at[pl.ds(base_e, gb), :], bufs_a[p],
                    gsem_a[p]).wait()
                pltpu.make_async_copy(
                    pre_h.at[pl.ds(base_e, gb), :], bufs_b[p],
                    gsem_b[p]).wait()

                ba, bb = bufs_a[p], bufs_b[p]

                def addrow(r, carry2):
                    for cc in range(D // 16):
                        sl = pl.ds(cc * 16, 16)
                        ba[r, sl] = ba[r, sl] + bb[r, sl]
                    return carry2

                lax.fori_loop(0, gb, addrow, 0)
                off = base_e + ch * gb
                pltpu.async_copy(ba, pre_h.at[pl.ds(off, gb), :], wsem[p])
            return carry

        lax.fori_loop(0, gc // _NB, body, 0)
        for ch in (gc - 3, gc - 2, gc - 1):
            sp = ch % _NB
            pltpu.make_async_copy(
                bufs_a[sp], pre_h.at[pl.ds(base_e, gb), :], wsem[sp]).wait()

    return gather_add


def _scatter_kernel(E, NP, D, gbs):
    ew = E // _NW
    gc = ew // gbs
    nt = NP // _NS
    assert gc % _NB == 0 and gbs % 8 == 0 and gbs <= 128
    mesh = plsc.VectorSubcoreMesh(core_axis_name="c", subcore_axis_name="s")

    @functools.partial(
        pl.kernel,
        out_type=jax.ShapeDtypeStruct((_NC, NP, D), _F32),
        mesh=mesh,
        scratch_types=[
            pltpu.VMEM((gc, gbs), jnp.int32),
            pltpu.VMEM_SHARED((NP, D), _F32),
        ] + [pltpu.VMEM((gbs, D), _F32)] * _NB
          + [pltpu.SemaphoreType.DMA] * _NB,
    )
    def scatter(edges_h, idx0_h, init_h, out_h, idx_v, agg_sh, *rest):
        bufs = rest[0:_NB]
        rsem = rest[_NB:2 * _NB]
        c = lax.axis_index("c")
        s = lax.axis_index("s")
        w = s * _NC + c
        base_e = w * ew
        pltpu.sync_copy(init_h.at[c, pl.ds(s * nt, nt), :],
                        agg_sh.at[pl.ds(s * nt, nt), :])
        pltpu.sync_copy(idx0_h.at[w], idx_v)
        plsc.subcore_barrier()

        pltpu.async_copy(edges_h.at[pl.ds(base_e, gbs), :], bufs[0], rsem[0])

        def body(i, carry):
            for p in range(_NB):
                ch = i * _NB + p
                sp = (p + 1) % _NB
                pltpu.make_async_copy(
                    edges_h.at[pl.ds(base_e, gbs), :], bufs[p],
                    rsem[p]).wait()

                @pl.when(ch + 1 < gc)
                def _():
                    pltpu.async_copy(
                        edges_h.at[pl.ds(base_e + (ch + 1) * gbs, gbs), :],
                        bufs[sp], rsem[sp])

                pltpu.sync_copy(bufs[p], agg_sh.at[idx_v.at[ch]], add=True)
            return carry

        lax.fori_loop(0, gc // _NB, body, 0)
        plsc.subcore_barrier()
        pltpu.sync_copy(agg_sh.at[pl.ds(s * nt, nt), :],
                        out_h.at[c, pl.ds(s * nt, nt), :])

    return scatter


def _premult_body(nd, w0r, w0s, out0, out1):
    out0[...] = jnp.dot(nd[...], w0r[...], preferred_element_type=_F32)
    out1[...] = jnp.dot(nd[...], w0s[...], preferred_element_type=_F32)


def _premult_call(N, D, H, tile):
    w = _full_spec((D, H))
    return pl.pallas_call(
        _premult_body,
        grid=(N // tile,),
        in_specs=[_row_spec(tile, D), w, w],
        out_specs=[_row_spec(tile, H), _row_spec(tile, H)],
        out_shape=[jax.ShapeDtypeStruct((N, H), _F32),
                   jax.ShapeDtypeStruct((N, H), _F32)],
    )


def _edge_mlp_body(ed, pre, w0e, b0, w1, b1, w2, b2, g, bn, out):
    x = jnp.dot(ed[...], w0e[...], preferred_element_type=_F32)
    x = x + pre[...]
    x = jax.nn.relu(x + b0[...])
    x = jax.nn.relu(jnp.dot(x, w1[...], preferred_element_type=_F32) + b1[...])
    y = jnp.dot(x, w2[...], preferred_element_type=_F32) + b2[...]
    mu = jnp.mean(y, axis=-1, keepdims=True)
    var = jnp.mean((y - mu) ** 2, axis=-1, keepdims=True)
    out[...] = (y - mu) / jnp.sqrt(var + 1e-5) * g[...] + bn[...]


def _node_mlp_body(pa, pb, nd, w0a, w0n, b0, w1, b1, w2, b2, g, bn, out):
    agg = pa[...] + pb[...]
    x = jnp.dot(agg, w0a[...], preferred_element_type=_F32)
    x += jnp.dot(nd[...], w0n[...], preferred_element_type=_F32)
    x = jax.nn.relu(x + b0[...])
    x = jax.nn.relu(jnp.dot(x, w1[...], preferred_element_type=_F32) + b1[...])
    y = jnp.dot(x, w2[...], preferred_element_type=_F32) + b2[...]
    mu = jnp.mean(y, axis=-1, keepdims=True)
    var = jnp.mean((y - mu) ** 2, axis=-1, keepdims=True)
    out[...] = (y - mu) / jnp.sqrt(var + 1e-5) * g[...] + bn[...]


def _row_spec(tile, d):
    return pl.BlockSpec((tile, d), lambda i: (i, 0))


def _full_spec(shape):
    return pl.BlockSpec(shape, lambda i: (0,) * len(shape))


def _edge_mlp_call(E, D, H, tile, ed_off=0):
    grid = (E // tile,)
    w = _full_spec((D, H))
    wh = _full_spec((H, H))
    w2 = _full_spec((H, D))
    b = _full_spec((1, H))
    bd = _full_spec((1, D))
    ed_spec = pl.BlockSpec((tile, D), lambda i: (i + ed_off, 0))
    return pl.pallas_call(
        _edge_mlp_body,
        grid=grid,
        in_specs=[ed_spec, _row_spec(tile, H)]
                 + [w, b, wh, b, w2, bd, bd, bd],
        out_specs=_row_spec(tile, D),
        out_shape=jax.ShapeDtypeStruct((E, D), _F32),
    )


def _node_mlp_call(N, D, H, tile):
    grid = (N // tile,)
    w = _full_spec((D, H))
    wh = _full_spec((H, H))
    w2 = _full_spec((H, D))
    b = _full_spec((1, H))
    bd = _full_spec((1, D))
    return pl.pallas_call(
        _node_mlp_body,
        grid=grid,
        in_specs=[_row_spec(tile, D)] * 3 + [w, w, b, wh, b, w2, bd, bd, bd],
        out_specs=_row_spec(tile, D),
        out_shape=jax.ShapeDtypeStruct((N, D), _F32),
    )


def kernel(nodes, edges, neighbor_idxs, params):
    N, D = nodes.shape
    E = edges.shape[0]
    L = params["edge_W0"].shape[0]
    H = params["edge_W1"].shape[1]

    E2 = E // 2
    GB = 40
    ew = E2 // _NW
    gc = ew // GB
    idx0 = neighbor_idxs[:, 0].astype(jnp.int32)
    idx1 = neighbor_idxs[:, 1].astype(jnp.int32)
    idx0_h = [idx0[h * E2:(h + 1) * E2].reshape(_NW, gc, GB) for h in range(2)]
    idx1_h = [idx1[h * E2:(h + 1) * E2].reshape(_NW, gc, GB) for h in range(2)]
    NP = ((N + 8 * _NS - 1) // (8 * _NS)) * (8 * _NS)
    zeros = jnp.zeros((_NC, NP, D), _F32)

    TILE = 2000
    gather_add = _gather_add_kernel(E2, D, GB)
    premult = _premult_call(N, D, H, tile=TILE)
    scatter = _scatter_kernel(E2, NP, D, GB)
    edge_mlp = _edge_mlp_call(E2, D, H, tile=TILE)
    edge_mlp_l0 = [_edge_mlp_call(E2, D, H, tile=TILE,
                                  ed_off=h * (E2 // TILE)) for h in range(2)]
    node_mlp = _node_mlp_call(N, D, H, tile=TILE)

    eh = [edges, edges]

    def b2d(b):
        return b.reshape(1, -1)

    for i in range(L):
        ew0 = params["edge_W0"][i]
        edge_args = (
            ew0[:D],
            b2d(params["edge_b0"][i]), params["edge_W1"][i],
            b2d(params["edge_b1"][i]), params["edge_W2"][i],
            b2d(params["edge_b2"][i]), b2d(params["edge_ln_g"][i]),
            b2d(params["edge_ln_b"][i]))
        pm0, pm1 = premult(nodes, ew0[D:2 * D], ew0[2 * D:])
        parts = zeros
        for h in range(2):
            pre = gather_add(pm0, pm1, idx0_h[h], idx1_h[h])
            mlp = edge_mlp_l0[h] if i == 0 else edge_mlp
            eh[h] = mlp(eh[h], pre, *edge_args)
            parts = scatter(eh[h], idx0_h[h], parts)
        nw0 = params["node_W0"][i]
        nodes = node_mlp(
            parts[0, :N], parts[1, :N], nodes,
            nw0[:D], nw0[D:],
            b2d(params["node_b0"][i]), params["node_W1"][i],
            b2d(params["node_b1"][i]), params["node_W2"][i],
            b2d(params["node_b2"][i]), b2d(params["node_ln_g"][i]),
            b2d(params["node_ln_b"][i]))

    return nodes, jnp.concatenate(eh, axis=0), neighbor_idxs
